# Initial kernel scaffold; baseline (speedup 1.0000x reference)
import jax
import jax.numpy as jnp
from jax.experimental import pallas as pl


def kernel(inputs, tables):
    raise NotImplementedError("write your pallas kernel here")


# speedup vs baseline: 5.1407x; 5.1407x over previous
import functools

import jax
import jax.numpy as jnp
from jax import lax
from jax.experimental import pallas as pl
from jax.experimental.pallas import tpu as pltpu
from jax.experimental.pallas import tpu_sc as plsc

F = 26
V = 50
D = 32
B = 16384

ROWS = B * F
NC = 2
NS = 16
NW = NC * NS
ROWS_PER_W = ROWS // NW
BLK = 128
NBLK = ROWS_PER_W // BLK
LANES = 16


def _sc_body(inputs_hbm, tables_hbm, out_hbm, idx_v, buf0, buf1, sem0, sem1):
    wid = lax.axis_index("s") * NC + lax.axis_index("c")

    pltpu.sync_copy(inputs_hbm.at[wid], idx_v)

    row_base = wid * ROWS_PER_W

    def compute_idx(j):
        for i in range(BLK // LANES):
            p = lax.iota(jnp.int32, LANES) + (row_base + j * BLK + i * LANES)
            sl = pl.ds(i * LANES, LANES)
            idx_v[j, sl] = idx_v[j, sl] + (p % F) * V

    def fire(j, buf, sem):
        pltpu.async_copy(tables_hbm.at[idx_v.at[j]], buf, sem)

    def drain(j, buf, sem):
        pltpu.make_async_copy(tables_hbm.at[idx_v.at[j]], buf, sem).wait()

    compute_idx(0)
    fire(0, buf0, sem0)

    def step(j, buf, sem, nbuf, nsem):
        @pl.when(j + 1 < NBLK)
        def _():
            compute_idx(j + 1)
            fire(j + 1, nbuf, nsem)

        drain(j, buf, sem)
        pltpu.sync_copy(buf, out_hbm.at[wid, j])

    def pair(t, carry):
        step(2 * t, buf0, sem0, buf1, sem1)
        step(2 * t + 1, buf1, sem1, buf0, sem0)
        return carry

    lax.fori_loop(0, NBLK // 2, pair, 0)


@jax.jit
def _lookup(inputs_flat, tables_flat):
    mesh = plsc.VectorSubcoreMesh(core_axis_name="c", subcore_axis_name="s")
    run = pl.kernel(
        _sc_body,
        out_type=jax.ShapeDtypeStruct((NW, NBLK, BLK, D), jnp.float32),
        mesh=mesh,
        scratch_types=[
            pltpu.VMEM((NBLK, BLK), jnp.int32),
            pltpu.VMEM((BLK, D), jnp.float32),
            pltpu.VMEM((BLK, D), jnp.float32),
            pltpu.SemaphoreType.DMA,
            pltpu.SemaphoreType.DMA,
        ],
        compiler_params=pltpu.CompilerParams(use_tc_tiling_on_sc=False),
    )
    return run(inputs_flat, tables_flat)


def kernel(inputs, tables):
    inputs_flat = inputs.reshape(NW, NBLK, BLK)
    tables_flat = tables.reshape(F * V, D)
    out = _lookup(inputs_flat, tables_flat)
    return out.reshape(B, F, D)
